# baseline (speedup 1.0000x reference)
import jax
import jax.numpy as jnp
import numpy as np
from jax import lax
from jax.experimental import pallas as pl
from jax.experimental.pallas import tpu as pltpu
from jax.experimental.pallas import tpu_sc as plsc

MAX_LEN = 200
EMBED_DIM = 128
BATCH = 4096

NUM_CORES = 2
NUM_SUBCORES = 16
NUM_WORKERS = NUM_CORES * NUM_SUBCORES
SEQS_PER_WORKER = BATCH // NUM_WORKERS
LANES = 16
VECS_PER_ROW = EMBED_DIM // LANES
NBUF = 4
NPAIR = SEQS_PER_WORKER // NBUF


def _make_pe_np():
    pos = np.arange(MAX_LEN, dtype=np.float64)[:, None]
    j = np.arange(EMBED_DIM, dtype=np.float64)[None, :]
    angle = pos / (10000.0 ** (j / float(EMBED_DIM)))
    pe = np.where((np.arange(EMBED_DIM)[None, :] % 2) == 0, np.sin(angle), np.cos(angle))
    return pe.astype(np.float32)


_PE = _make_pe_np()


def _sc_body(x_hbm, table_hbm, pe_hbm, out_hbm,
             idx0, idx1, idx2, idx3, rows0, rows1, rows2, rows3, pe_v,
             gsem0, gsem1, gsem2, gsem3,
             isem0, isem1, isem2, isem3,
             osem0, osem1, osem2, osem3):
    idx = (idx0, idx1, idx2, idx3)
    rows = (rows0, rows1, rows2, rows3)
    gsem = (gsem0, gsem1, gsem2, gsem3)
    isem = (isem0, isem1, isem2, isem3)
    osem = (osem0, osem1, osem2, osem3)

    wid = lax.axis_index("s") * NUM_CORES + lax.axis_index("c")
    seq0 = wid * SEQS_PER_WORKER

    pltpu.sync_copy(pe_hbm, pe_v)

    def idx_copy(j, b):
        row0 = (seq0 + j) * MAX_LEN
        return pltpu.make_async_copy(x_hbm.at[pl.ds(row0, MAX_LEN)], idx[b], isem[b])

    def gather(b):
        return pltpu.make_async_copy(table_hbm.at[idx[b]], rows[b], gsem[b])

    def store(j, b):
        row0 = (seq0 + j) * MAX_LEN
        return pltpu.make_async_copy(rows[b], out_hbm.at[pl.ds(row0, MAX_LEN)], osem[b])

    for b in range(NBUF):
        pltpu.sync_copy(x_hbm.at[pl.ds((seq0 + b) * MAX_LEN, MAX_LEN)], idx[b])
    gather(0).start()
    gather(1).start()

    def quad(k, carry):
        for b in range(NBUF):
            j = NBUF * k + b
            bg = (b + 2) % NBUF
            gather(b).wait()

            @pl.when(k <= NPAIR - 2)
            def _():
                idx_copy(j + 4, b).start()

            def per_row(r, c2):
                for c in range(VECS_PER_ROW):
                    sl = pl.ds(c * LANES, LANES)
                    plsc.addupdate(rows[b].at[r, sl], pe_v[r, sl])
                return c2

            lax.fori_loop(0, MAX_LEN, per_row, 0)
            store(j, b).start()

            if b < 2:
                @pl.when(k >= 1)
                def _():
                    store(j - 2, bg).wait()
                    idx_copy(j + 2, bg).wait()

                gather(bg).start()
            else:
                @pl.when(k <= NPAIR - 2)
                def _():
                    store(j - 2, bg).wait()
                    idx_copy(j + 2, bg).wait()
                    gather(bg).start()
        return carry

    lax.fori_loop(0, NPAIR, quad, 0)

    for b in range(NBUF):
        store(SEQS_PER_WORKER - NBUF + b, b).wait()


@jax.jit
def _pos_embed(x_flat, table, pe):
    mesh = plsc.VectorSubcoreMesh(core_axis_name="c", subcore_axis_name="s")
    return pl.kernel(
        _sc_body,
        out_type=jax.ShapeDtypeStruct((BATCH * MAX_LEN, EMBED_DIM), jnp.float32),
        mesh=mesh,
        scratch_types=(
            [pltpu.VMEM((MAX_LEN,), jnp.int32) for _ in range(NBUF)]
            + [pltpu.VMEM((MAX_LEN, EMBED_DIM), jnp.float32) for _ in range(NBUF)]
            + [pltpu.VMEM((MAX_LEN, EMBED_DIM), jnp.float32)]
            + [pltpu.SemaphoreType.DMA for _ in range(3 * NBUF)]
        ),
    )(x_flat, table, pe)


def kernel(x, embed_weight):
    x_flat = x.reshape(-1).astype(jnp.int32)
    pe = jnp.asarray(_PE)
    out = _pos_embed(x_flat, embed_weight, pe)
    return out.reshape(BATCH, MAX_LEN, EMBED_DIM)
